# Initial kernel scaffold; baseline (speedup 1.0000x reference)
import jax
import jax.numpy as jnp
from jax.experimental import pallas as pl


def kernel(x, W, b, embed, gumbel):
    raise NotImplementedError("write your pallas kernel here")


# speedup vs baseline: 2.8280x; 2.8280x over previous
import functools

import jax
import jax.numpy as jnp
from jax.experimental import pallas as pl
from jax.experimental.pallas import tpu as pltpu
from jax.experimental.pallas import tpu_sc as plsc

N = 4608
IN_FEATURES = 768
EMBED_ENTRIES = 8192
EMBED_DIM = 256

BLOCK_N = 512
BLOCK_M = 1024
GRID_N = N // BLOCK_N
GRID_M = EMBED_ENTRIES // BLOCK_M
LOGK = float(jnp.log(jnp.float32(EMBED_ENTRIES)))

GATHER_WINDOW = 128


def _tc_body(x_ref, w_ref, b_ref, g_ref, idx_ref, diff_ref,
             zmax_ref, zidx_ref, m_ref, s_ref, t_ref):
    i = pl.program_id(0)
    j = pl.program_id(1)

    l = jax.lax.dot_general(
        x_ref[...], w_ref[...],
        dimension_numbers=(((1,), (1,)), ((), ())),
        preferred_element_type=jnp.float32,
        precision=jax.lax.Precision.DEFAULT,
    ) + b_ref[...]
    z = l + g_ref[...]

    iota = jax.lax.broadcasted_iota(jnp.int32, z.shape, 1) + j * BLOCK_M
    bm = jnp.max(z, axis=1, keepdims=True)
    bidx = jnp.min(jnp.where(z == bm, iota, jnp.int32(2**30)),
                   axis=1, keepdims=True)
    lm = jnp.max(l, axis=1, keepdims=True)

    @pl.when(j == 0)
    def _():
        zmax_ref[...] = bm
        zidx_ref[...] = bidx
        m_ref[...] = lm
        e = jnp.exp(l - lm)
        s_ref[...] = jnp.sum(e, axis=1, keepdims=True)
        t_ref[...] = jnp.sum(e * l, axis=1, keepdims=True)

    @pl.when(j > 0)
    def _():
        better = bm > zmax_ref[...]
        zmax_ref[...] = jnp.where(better, bm, zmax_ref[...])
        zidx_ref[...] = jnp.where(better, bidx, zidx_ref[...])
        new_m = jnp.maximum(m_ref[...], lm)
        scale = jnp.exp(m_ref[...] - new_m)
        e = jnp.exp(l - new_m)
        s_ref[...] = s_ref[...] * scale + jnp.sum(e, axis=1, keepdims=True)
        t_ref[...] = t_ref[...] * scale + jnp.sum(e * l, axis=1, keepdims=True)
        m_ref[...] = new_m

    @pl.when(j == GRID_M - 1)
    def _():
        idx_ref[...] = zidx_ref[...].reshape(1, 1, BLOCK_N)
        s = s_ref[...]
        drow = t_ref[...] / s - (m_ref[...] + jnp.log(s)) + LOGK
        part = (jnp.sum(drow) / N).reshape(1, 1)

        @pl.when(i == 0)
        def _():
            diff_ref[...] = part

        @pl.when(i > 0)
        def _():
            diff_ref[...] = diff_ref[...] + part


@functools.partial(jax.jit, static_argnames=("interpret",))
def _tc_part(x, W, b2d, gumbel, interpret=False):
    return pl.pallas_call(
        _tc_body,
        grid=(GRID_N, GRID_M),
        in_specs=[
            pl.BlockSpec((BLOCK_N, IN_FEATURES), lambda i, j: (i, 0)),
            pl.BlockSpec((BLOCK_M, IN_FEATURES), lambda i, j: (j, 0)),
            pl.BlockSpec((1, BLOCK_M), lambda i, j: (0, j)),
            pl.BlockSpec((BLOCK_N, BLOCK_M), lambda i, j: (i, j)),
        ],
        out_specs=[
            pl.BlockSpec((1, 1, BLOCK_N), lambda i, j: (i, 0, 0)),
            pl.BlockSpec((1, 1), lambda i, j: (0, 0)),
        ],
        out_shape=[
            jax.ShapeDtypeStruct((GRID_N, 1, BLOCK_N), jnp.int32),
            jax.ShapeDtypeStruct((1, 1), jnp.float32),
        ],
        scratch_shapes=[
            pltpu.VMEM((BLOCK_N, 1), jnp.float32),
            pltpu.VMEM((BLOCK_N, 1), jnp.int32),
            pltpu.VMEM((BLOCK_N, 1), jnp.float32),
            pltpu.VMEM((BLOCK_N, 1), jnp.float32),
            pltpu.VMEM((BLOCK_N, 1), jnp.float32),
        ],
        interpret=interpret,
    )(x, W, b2d, gumbel)


def _sc_gather(embed, idx):
    idx2 = idx.reshape(1, N)

    @functools.partial(
        pl.kernel,
        out_type=jax.ShapeDtypeStruct((N, EMBED_DIM), embed.dtype),
        mesh=plsc.VectorSubcoreMesh(core_axis_name="core",
                                    subcore_axis_name="subcore"),
    )
    def kern(x_hbm, i_hbm, o_hbm):
        def body(i_vmem, o_vmem):
            pltpu.sync_copy(x_hbm.at[i_vmem.at[0]], o_vmem)

        pltpu.emit_pipeline(
            body,
            grid=(N // GATHER_WINDOW,),
            in_specs=[pl.BlockSpec((1, GATHER_WINDOW),
                                   index_map=lambda i: (0, i))],
            out_specs=[pl.BlockSpec((GATHER_WINDOW, EMBED_DIM),
                                    index_map=lambda i: (i, 0))],
            core_axis_name=("core", "subcore"),
            dimension_semantics=(pltpu.PARALLEL,),
        )(i_hbm, o_hbm)

    return kern(embed, idx2)


def kernel(x, W, b, embed, gumbel):
    idx3d, diff2d = _tc_part(x, W, b.reshape(1, -1), gumbel)
    idx = idx3d.reshape(N)
    quantize = _sc_gather(embed, idx)
    return (quantize, diff2d.reshape(()), idx)
